# Initial kernel scaffold; baseline (speedup 1.0000x reference)
import functools

import jax
import jax.numpy as jnp
from jax import lax
from jax.experimental import pallas as pl
from jax.experimental.pallas import tpu as pltpu
from jax.experimental.pallas import tpu_sc as plsc

N_PREFIX = 10000
N_AS = 10000
N_EDGES = 320000
EMBED = 64

NC = 2
NS = 16
NW = NC * NS
CH = 128
CPT = 80
E_PAD = NW * CPT * CH
N_ACC = 10240
ROWS_PT = N_ACC // NS


def _fmap_body(x_as_ref, w_as_ref, b_as_ref, w_att_ref, a2_ref,
               as_out_ref, g_ref):
    x = x_as_ref[...]
    h = x @ w_as_ref[...] + b_as_ref[...]
    h = jnp.maximum(h, 0.0)
    nrm = jnp.sqrt(jnp.sum(h * h, axis=1, keepdims=True))
    as_out_ref[...] = h / jnp.maximum(nrm, 1e-12)
    w2 = w_att_ref[...] @ a2_ref[...]
    s = h @ w2
    es = jnp.exp(s - jnp.max(s))
    g_ref[...] = es * h


def _fmap_call(x_as, w_as, b_as, w_att, a2):
    return pl.pallas_call(
        _fmap_body,
        out_shape=(
            jax.ShapeDtypeStruct((N_AS, EMBED), jnp.float32),
            jax.ShapeDtypeStruct((N_AS, EMBED), jnp.float32),
        ),
    )(x_as, w_as, b_as, w_att, a2)


_MESH = plsc.VectorSubcoreMesh(core_axis_name="c", subcore_axis_name="s")


@functools.partial(
    pl.kernel,
    out_type=jax.ShapeDtypeStruct((NC, N_ACC, EMBED), jnp.float32),
    mesh=_MESH,
    scratch_types=[
        pltpu.VMEM((CPT, CH), jnp.int32),
        pltpu.VMEM((CPT, CH), jnp.int32),
        pltpu.VMEM((CH, EMBED), jnp.float32),
        pltpu.VMEM_SHARED((N_ACC, EMBED), jnp.float32),
        pltpu.SemaphoreType.DMA,
    ],
)
def _edge_kernel(u2d, v2d, g, zeros_hbm, out,
                 vidx, uidx, rows, acc, sem):
    c = lax.axis_index("c")
    s = lax.axis_index("s")
    wid = c * NS + s
    row0 = wid * CPT
    pltpu.sync_copy(v2d.at[pl.ds(row0, CPT)], vidx)
    pltpu.sync_copy(u2d.at[pl.ds(row0, CPT)], uidx)
    pltpu.sync_copy(zeros_hbm.at[pl.ds(s * ROWS_PT, ROWS_PT)],
                    acc.at[pl.ds(s * ROWS_PT, ROWS_PT)])
    plsc.subcore_barrier()

    def body(k, carry):
        pltpu.async_copy(g.at[vidx.at[k]], rows, sem).wait()
        pltpu.sync_copy(rows, acc.at[uidx.at[k]], add=True)
        return carry

    lax.fori_loop(0, CPT, body, 0)
    plsc.subcore_barrier()
    pltpu.sync_copy(acc.at[pl.ds(s * ROWS_PT, ROWS_PT)],
                    out.at[c, pl.ds(s * ROWS_PT, ROWS_PT)])


def _norm_body(acc_ref, out_ref):
    p = acc_ref[0] + acc_ref[1]
    nrm = jnp.sqrt(jnp.sum(p * p, axis=1, keepdims=True))
    out_ref[...] = p / jnp.maximum(nrm, 1e-12)


def _norm_call(acc2):
    return pl.pallas_call(
        _norm_body,
        out_shape=jax.ShapeDtypeStruct((N_ACC, EMBED), jnp.float32),
    )(acc2)


def kernel(x_prefix, x_as, W_prefix, b_prefix, W_as, b_as,
           W_att, b_att, a_att, b_a, edge_u, edge_v):
    del x_prefix, W_prefix, b_prefix, b_att, b_a
    a2 = a_att[EMBED:, :]
    as_out, g_as = _fmap_call(x_as, W_as, b_as.reshape(1, EMBED), W_att, a2)

    pad = E_PAD - N_EDGES
    u_p = jnp.concatenate(
        [edge_u, jnp.full((pad,), N_PREFIX, jnp.int32)]).reshape(-1, CH)
    v_p = jnp.concatenate(
        [edge_v, jnp.zeros((pad,), jnp.int32)]).reshape(-1, CH)
    zeros = jnp.zeros((N_ACC, EMBED), jnp.float32)

    acc2 = _edge_kernel(u_p, v_p, g_as, zeros)
    prefix_full = _norm_call(acc2)
    return prefix_full[:N_PREFIX], as_out


# speedup vs baseline: 22.0866x; 22.0866x over previous
import functools

import jax
import jax.numpy as jnp
from jax import lax
from jax.experimental import pallas as pl
from jax.experimental.pallas import tpu as pltpu
from jax.experimental.pallas import tpu_sc as plsc

N_PREFIX = 10000
N_AS = 10000
N_EDGES = 320000
EMBED = 64

NC = 2
NS = 16
NW = NC * NS
CH = 128
CPT = 80
E_PAD = NW * CPT * CH
N_ACC = 10240
ROWS_PT = N_ACC // NS


def _fmap_body(x_as_ref, w_as_ref, b_as_ref, w_att_ref, a2_ref,
               as_out_ref, g_ref):
    x = x_as_ref[...]
    h = x @ w_as_ref[...] + b_as_ref[...]
    h = jnp.maximum(h, 0.0)
    nrm = jnp.sqrt(jnp.sum(h * h, axis=1, keepdims=True))
    as_out_ref[...] = h / jnp.maximum(nrm, 1e-12)
    w2 = w_att_ref[...] @ a2_ref[...]
    s = h @ w2
    es = jnp.exp(s - jnp.max(s))
    g_ref[...] = es * h


def _fmap_call(x_as, w_as, b_as, w_att, a2):
    return pl.pallas_call(
        _fmap_body,
        out_shape=(
            jax.ShapeDtypeStruct((N_AS, EMBED), jnp.float32),
            jax.ShapeDtypeStruct((N_AS, EMBED), jnp.float32),
        ),
    )(x_as, w_as, b_as, w_att, a2)


_MESH = plsc.VectorSubcoreMesh(core_axis_name="c", subcore_axis_name="s")


@functools.partial(
    pl.kernel,
    out_type=jax.ShapeDtypeStruct((NC, N_ACC, EMBED), jnp.float32),
    mesh=_MESH,
    compiler_params=pltpu.CompilerParams(use_tc_tiling_on_sc=False),
    scratch_types=[
        pltpu.VMEM((CPT, CH), jnp.int32),
        pltpu.VMEM((CPT, CH), jnp.int32),
        pltpu.VMEM((CH, EMBED), jnp.float32),
        pltpu.VMEM_SHARED((N_ACC, EMBED), jnp.float32),
        pltpu.SemaphoreType.DMA,
    ],
)
def _edge_kernel(u2d, v2d, g, zeros_hbm, out,
                 vidx, uidx, rows, acc, sem):
    c = lax.axis_index("c")
    s = lax.axis_index("s")
    wid = c * NS + s
    row0 = wid * CPT
    pltpu.sync_copy(v2d.at[pl.ds(row0, CPT)], vidx)
    pltpu.sync_copy(u2d.at[pl.ds(row0, CPT)], uidx)
    pltpu.sync_copy(zeros_hbm.at[pl.ds(s * ROWS_PT, ROWS_PT)],
                    acc.at[pl.ds(s * ROWS_PT, ROWS_PT)])
    plsc.subcore_barrier()

    def body(k, carry):
        pltpu.async_copy(g.at[vidx.at[k]], rows, sem).wait()
        pltpu.sync_copy(rows, acc.at[uidx.at[k]], add=True)
        return carry

    lax.fori_loop(0, CPT, body, 0)
    plsc.subcore_barrier()
    pltpu.sync_copy(acc.at[pl.ds(s * ROWS_PT, ROWS_PT)],
                    out.at[c, pl.ds(s * ROWS_PT, ROWS_PT)])


def _norm_body(acc_ref, out_ref):
    p = acc_ref[0] + acc_ref[1]
    nrm = jnp.sqrt(jnp.sum(p * p, axis=1, keepdims=True))
    out_ref[...] = p / jnp.maximum(nrm, 1e-12)


def _norm_call(acc2):
    return pl.pallas_call(
        _norm_body,
        out_shape=jax.ShapeDtypeStruct((N_ACC, EMBED), jnp.float32),
    )(acc2)


def kernel(x_prefix, x_as, W_prefix, b_prefix, W_as, b_as,
           W_att, b_att, a_att, b_a, edge_u, edge_v):
    del x_prefix, W_prefix, b_prefix, b_att, b_a
    a2 = a_att[EMBED:, :]
    as_out, g_as = _fmap_call(x_as, W_as, b_as.reshape(1, EMBED), W_att, a2)

    pad = E_PAD - N_EDGES
    u_p = jnp.concatenate(
        [edge_u, jnp.full((pad,), N_PREFIX, jnp.int32)]).reshape(-1, CH)
    v_p = jnp.concatenate(
        [edge_v, jnp.zeros((pad,), jnp.int32)]).reshape(-1, CH)
    zeros = jnp.zeros((N_ACC, EMBED), jnp.float32)

    acc2 = _edge_kernel(u_p, v_p, g_as, zeros)
    prefix_full = _norm_call(acc2)
    return prefix_full[:N_PREFIX], as_out


# speedup vs baseline: 26.1228x; 1.1827x over previous
import functools

import jax
import jax.numpy as jnp
from jax import lax
from jax.experimental import pallas as pl
from jax.experimental.pallas import tpu as pltpu
from jax.experimental.pallas import tpu_sc as plsc

N_PREFIX = 10000
N_AS = 10000
N_EDGES = 320000
EMBED = 64

NC = 2
NS = 16
NW = NC * NS
CH = 128
CPT = 80
E_PAD = NW * CPT * CH
N_ACC = 10240
ROWS_PT = N_ACC // NS
NBUF = 8
SLAG = 4


def _fmap_body(x_as_ref, w_as_ref, b_as_ref, w_att_ref, a2_ref,
               as_out_ref, g_ref):
    x = x_as_ref[...]
    h = x @ w_as_ref[...] + b_as_ref[...]
    h = jnp.maximum(h, 0.0)
    nrm = jnp.sqrt(jnp.sum(h * h, axis=1, keepdims=True))
    as_out_ref[...] = h / jnp.maximum(nrm, 1e-12)
    w2 = w_att_ref[...] @ a2_ref[...]
    s = h @ w2
    es = jnp.exp(s - jnp.max(s))
    g_ref[...] = es * h


def _fmap_call(x_as, w_as, b_as, w_att, a2):
    return pl.pallas_call(
        _fmap_body,
        out_shape=(
            jax.ShapeDtypeStruct((N_AS, EMBED), jnp.float32),
            jax.ShapeDtypeStruct((N_AS, EMBED), jnp.float32),
        ),
    )(x_as, w_as, b_as, w_att, a2)


_MESH = plsc.VectorSubcoreMesh(core_axis_name="c", subcore_axis_name="s")


@functools.partial(
    pl.kernel,
    out_type=jax.ShapeDtypeStruct((NC, N_ACC, EMBED), jnp.float32),
    mesh=_MESH,
    compiler_params=pltpu.CompilerParams(use_tc_tiling_on_sc=False),
    scratch_types=[
        pltpu.VMEM((CPT, CH), jnp.int32),
        pltpu.VMEM((CPT, CH), jnp.int32),
        pltpu.VMEM((NBUF, CH, EMBED), jnp.float32),
        pltpu.VMEM_SHARED((N_ACC, EMBED), jnp.float32),
        pltpu.SemaphoreType.DMA,
        pltpu.SemaphoreType.DMA,
    ],
)
def _edge_kernel(u2d, v2d, g, zeros_hbm, out,
                 vidx, uidx, rows, acc, gsem, ssem):
    c = lax.axis_index("c")
    s = lax.axis_index("s")
    wid = c * NS + s
    row0 = wid * CPT
    pltpu.sync_copy(v2d.at[pl.ds(row0, CPT)], vidx)
    pltpu.sync_copy(u2d.at[pl.ds(row0, CPT)], uidx)
    pltpu.sync_copy(zeros_hbm.at[pl.ds(s * ROWS_PT, ROWS_PT)],
                    acc.at[pl.ds(s * ROWS_PT, ROWS_PT)])
    plsc.subcore_barrier()

    for j in range(SLAG):
        pltpu.async_copy(g.at[vidx.at[j]], rows.at[j], gsem)

    def body(i, carry):
        b = lax.rem(i, NBUF)
        pltpu.make_async_copy(g.at[vidx.at[i]], rows.at[b], gsem).wait()
        pltpu.async_copy(rows.at[b], acc.at[uidx.at[i]], ssem, add=True)

        @pl.when(i >= SLAG)
        def _():
            bm = lax.rem(i - SLAG, NBUF)
            pltpu.make_async_copy(rows.at[bm], acc.at[uidx.at[i - SLAG]],
                                  ssem).wait()

        @pl.when(i + SLAG < CPT)
        def _():
            bn = lax.rem(i + SLAG, NBUF)
            pltpu.async_copy(g.at[vidx.at[i + SLAG]], rows.at[bn], gsem)

        return carry

    lax.fori_loop(0, CPT, body, 0)
    for j in range(CPT - SLAG, CPT):
        pltpu.make_async_copy(rows.at[j % NBUF], acc.at[uidx.at[j]],
                              ssem).wait()
    plsc.subcore_barrier()
    pltpu.sync_copy(acc.at[pl.ds(s * ROWS_PT, ROWS_PT)],
                    out.at[c, pl.ds(s * ROWS_PT, ROWS_PT)])


def _norm_body(acc_ref, out_ref):
    p = acc_ref[0] + acc_ref[1]
    nrm = jnp.sqrt(jnp.sum(p * p, axis=1, keepdims=True))
    out_ref[...] = p / jnp.maximum(nrm, 1e-12)


def _norm_call(acc2):
    return pl.pallas_call(
        _norm_body,
        out_shape=jax.ShapeDtypeStruct((N_ACC, EMBED), jnp.float32),
    )(acc2)


def kernel(x_prefix, x_as, W_prefix, b_prefix, W_as, b_as,
           W_att, b_att, a_att, b_a, edge_u, edge_v):
    del x_prefix, W_prefix, b_prefix, b_att, b_a
    a2 = a_att[EMBED:, :]
    as_out, g_as = _fmap_call(x_as, W_as, b_as.reshape(1, EMBED), W_att, a2)

    pad = E_PAD - N_EDGES
    u_p = jnp.concatenate(
        [edge_u, jnp.full((pad,), N_PREFIX, jnp.int32)]).reshape(-1, CH)
    v_p = jnp.concatenate(
        [edge_v, jnp.zeros((pad,), jnp.int32)]).reshape(-1, CH)
    zeros = jnp.zeros((N_ACC, EMBED), jnp.float32)

    acc2 = _edge_kernel(u_p, v_p, g_as, zeros)
    prefix_full = _norm_call(acc2)
    return prefix_full[:N_PREFIX], as_out
